# Initial kernel scaffold; baseline (speedup 1.0000x reference)
import functools

import jax
import jax.numpy as jnp
from jax import lax
from jax.experimental import pallas as pl
from jax.experimental.pallas import tpu as pltpu
from jax.experimental.pallas import tpu_sc as plsc

VOCAB = 1_000_000
D = 128
B_TOTAL = 16384 * 50
NW = 32
B_PER_W = B_TOTAL // NW
CHUNK = 128
NCHUNK = B_PER_W // CHUNK


def _gather_kernel(table_hbm, idx_hbm, out_hbm, idx_v, rows_v, sem):
    wid = lax.axis_index("s") * 2 + lax.axis_index("c")
    base = wid * B_PER_W

    def body(g, carry):
        off = base + g * CHUNK
        pltpu.sync_copy(idx_hbm.at[pl.ds(off, CHUNK)], idx_v)
        pltpu.async_copy(table_hbm.at[idx_v], rows_v, sem).wait()
        pltpu.sync_copy(rows_v, out_hbm.at[pl.ds(off, CHUNK)])
        return carry

    lax.fori_loop(0, NCHUNK, body, 0)


def kernel(indices, embeddings):
    idx = indices.reshape(-1).astype(jnp.int32)
    mesh = plsc.VectorSubcoreMesh(core_axis_name="c", subcore_axis_name="s")
    run = functools.partial(
        pl.kernel,
        mesh=mesh,
        out_type=jax.ShapeDtypeStruct((B_TOTAL, D), jnp.float32),
        scratch_types=[
            pltpu.VMEM((CHUNK,), jnp.int32),
            pltpu.VMEM((CHUNK, D), jnp.float32),
            pltpu.SemaphoreType.DMA,
        ],
    )(_gather_kernel)
    out = run(embeddings, idx)
    return out.reshape(indices.shape + (D,))


# speedup vs baseline: 1.0418x; 1.0418x over previous
import functools

import jax
import jax.numpy as jnp
from jax import lax
from jax.experimental import pallas as pl
from jax.experimental.pallas import tpu as pltpu
from jax.experimental.pallas import tpu_sc as plsc

VOCAB = 1_000_000
D = 128
B_TOTAL = 16384 * 50
NW = 32
B_PER_W = B_TOTAL // NW
CHUNK = 128
NCHUNK = B_PER_W // CHUNK


def _gather_kernel(table_hbm, idx_hbm, out_hbm, idx_v, rows_v, sem):
    wid = lax.axis_index("s") * 2 + lax.axis_index("c")

    def body(g, carry):
        pltpu.sync_copy(idx_hbm.at[wid, g], idx_v)
        pltpu.async_copy(table_hbm.at[idx_v], rows_v, sem).wait()
        pltpu.sync_copy(rows_v, out_hbm.at[wid, g])
        return carry

    lax.fori_loop(0, NCHUNK, body, 0)


def kernel(indices, embeddings):
    idx = indices.reshape(NW, NCHUNK, CHUNK).astype(jnp.int32)
    mesh = plsc.VectorSubcoreMesh(core_axis_name="c", subcore_axis_name="s")
    run = functools.partial(
        pl.kernel,
        mesh=mesh,
        out_type=jax.ShapeDtypeStruct((NW, NCHUNK, CHUNK, D), jnp.float32),
        scratch_types=[
            pltpu.VMEM((CHUNK,), jnp.int32),
            pltpu.VMEM((CHUNK, D), jnp.float32),
            pltpu.SemaphoreType.DMA,
        ],
    )(_gather_kernel)
    out = run(embeddings, idx)
    return out.reshape(indices.shape + (D,))


# speedup vs baseline: 1.1994x; 1.1513x over previous
import functools

import jax
import jax.numpy as jnp
from jax import lax
from jax.experimental import pallas as pl
from jax.experimental.pallas import tpu as pltpu
from jax.experimental.pallas import tpu_sc as plsc

VOCAB = 1_000_000
D = 128
B_TOTAL = 16384 * 50
NW = 32
B_PER_W = B_TOTAL // NW
CHUNK = 128
NCHUNK = B_PER_W // CHUNK
NB = 4
NOUTER = NCHUNK // NB


def _gather_kernel(table_hbm, idx_hbm, out_hbm, idx_full, rows_v,
                   gs0, gs1, gs2, gs3, os0, os1, os2, os3):
    gsems = (gs0, gs1, gs2, gs3)
    osems = (os0, os1, os2, os3)
    wid = lax.axis_index("s") * 2 + lax.axis_index("c")

    pltpu.sync_copy(idx_hbm.at[wid], idx_full)

    def body(t, carry):
        for b in range(NB):
            g = t * NB + b

            @pl.when(t > 0)
            def _wait_prev_out():
                pltpu.make_async_copy(rows_v.at[b], out_hbm.at[wid, g],
                                      osems[b]).wait()

            pltpu.async_copy(table_hbm.at[idx_full.at[g]], rows_v.at[b],
                             gsems[b]).wait()
            pltpu.make_async_copy(rows_v.at[b], out_hbm.at[wid, g],
                                  osems[b]).start()
        return carry

    lax.fori_loop(0, NOUTER, body, 0)

    for b in range(NB):
        g = (NOUTER - 1) * NB + b
        pltpu.make_async_copy(rows_v.at[b], out_hbm.at[wid, g],
                              osems[b]).wait()


def kernel(indices, embeddings):
    idx = indices.reshape(NW, NCHUNK, CHUNK).astype(jnp.int32)
    mesh = plsc.VectorSubcoreMesh(core_axis_name="c", subcore_axis_name="s")
    run = functools.partial(
        pl.kernel,
        mesh=mesh,
        out_type=jax.ShapeDtypeStruct((NW, NCHUNK, CHUNK, D), jnp.float32),
        scratch_types=[
            pltpu.VMEM((NCHUNK, CHUNK), jnp.int32),
            pltpu.VMEM((NB, CHUNK, D), jnp.float32),
        ] + [pltpu.SemaphoreType.DMA] * (2 * NB),
    )(_gather_kernel)
    out = run(embeddings, idx)
    return out.reshape(indices.shape + (D,))


# speedup vs baseline: 1.2788x; 1.0662x over previous
import functools

import jax
import jax.numpy as jnp
from jax import lax
from jax.experimental import pallas as pl
from jax.experimental.pallas import tpu as pltpu
from jax.experimental.pallas import tpu_sc as plsc

VOCAB = 1_000_000
D = 128
B_TOTAL = 16384 * 50
NW = 32
B_PER_W = B_TOTAL // NW
CHUNK = 128
NCHUNK = B_PER_W // CHUNK
NB = 5
A = 3
NOUTER = NCHUNK // NB
assert NOUTER * NB == NCHUNK


def _gather_kernel(table_hbm, idx_hbm, out_hbm, idx_full, rows_v, *sems):
    gsems = sems[:NB]
    osems = sems[NB:]
    wid = lax.axis_index("s") * 2 + lax.axis_index("c")

    pltpu.sync_copy(idx_hbm.at[wid], idx_full)

    def start_gather(g, b):
        pltpu.make_async_copy(table_hbm.at[idx_full.at[g]], rows_v.at[b],
                              gsems[b]).start()

    def wait_gather(g, b):
        pltpu.make_async_copy(table_hbm.at[idx_full.at[g]], rows_v.at[b],
                              gsems[b]).wait()

    def start_out(g, b):
        pltpu.make_async_copy(rows_v.at[b], out_hbm.at[wid, g],
                              osems[b]).start()

    def wait_out(g, b):
        pltpu.make_async_copy(rows_v.at[b], out_hbm.at[wid, g],
                              osems[b]).wait()

    for g in range(A):
        start_gather(g, g % NB)

    def body(t, carry):
        for b0 in range(NB):
            g = t * NB + b0
            wait_gather(g, b0)
            start_out(g, b0)
            ga = g + A
            ba = (b0 + A) % NB

            @pl.when(ga < NCHUNK)
            def _issue_ahead():
                @pl.when(ga >= NB)
                def _reuse_guard():
                    wait_out(ga - NB, ba)
                start_gather(ga, ba)
        return carry

    lax.fori_loop(0, NOUTER, body, 0)

    for b in range(NB):
        g = NCHUNK - NB + b
        wait_out(g, g % NB)


def kernel(indices, embeddings):
    idx = indices.reshape(NW, NCHUNK, CHUNK).astype(jnp.int32)
    mesh = plsc.VectorSubcoreMesh(core_axis_name="c", subcore_axis_name="s")
    run = functools.partial(
        pl.kernel,
        mesh=mesh,
        out_type=jax.ShapeDtypeStruct((NW, NCHUNK, CHUNK, D), jnp.float32),
        scratch_types=[
            pltpu.VMEM((NCHUNK, CHUNK), jnp.int32),
            pltpu.VMEM((NB, CHUNK, D), jnp.float32),
        ] + [pltpu.SemaphoreType.DMA] * (2 * NB),
    )(_gather_kernel)
    out = run(embeddings, idx)
    return out.reshape(indices.shape + (D,))


# speedup vs baseline: 2.0678x; 1.6169x over previous
import functools

import jax
import jax.numpy as jnp
from jax import lax
from jax.experimental import pallas as pl
from jax.experimental.pallas import tpu as pltpu
from jax.experimental.pallas import tpu_sc as plsc

VOCAB = 1_000_000
D = 128
SEQ = 50
NBATCH = 16384
NW = 32
BPW = NBATCH // NW
CB = 2
NCHUNK = BPW // CB
NB = 4
A = 3
NOUTER = NCHUNK // NB
assert NOUTER * NB == NCHUNK


def _gather_kernel(table_hbm, idx_hbm, out_hbm, idx_full, rows_v, *sems):
    gsems = sems[:NB]
    osems = sems[NB:]
    wid = lax.axis_index("s") * 2 + lax.axis_index("c")

    pltpu.sync_copy(idx_hbm.at[wid], idx_full)

    def start_gather(g, b):
        pltpu.make_async_copy(table_hbm.at[idx_full.at[g]], rows_v.at[b],
                              gsems[b]).start()

    def wait_gather(g, b):
        pltpu.make_async_copy(table_hbm.at[idx_full.at[g]], rows_v.at[b],
                              gsems[b]).wait()

    def start_out(g, b):
        for c in range(CB):
            pltpu.make_async_copy(rows_v.at[b, pl.ds(c * SEQ, SEQ)],
                                  out_hbm.at[wid, g, c], osems[b]).start()

    def wait_out(g, b):
        for c in range(CB):
            pltpu.make_async_copy(rows_v.at[b, pl.ds(c * SEQ, SEQ)],
                                  out_hbm.at[wid, g, c], osems[b]).wait()

    for g in range(A):
        start_gather(g, g % NB)

    def body(t, carry):
        for b0 in range(NB):
            g = t * NB + b0
            wait_gather(g, b0)
            start_out(g, b0)
            ga = g + A
            ba = (b0 + A) % NB

            @pl.when(ga < NCHUNK)
            def _issue_ahead():
                @pl.when(ga >= NB)
                def _reuse_guard():
                    wait_out(ga - NB, ba)
                start_gather(ga, ba)
        return carry

    lax.fori_loop(0, NOUTER, body, 0)

    for b in range(NB):
        g = NCHUNK - NB + b
        wait_out(g, g % NB)


def kernel(indices, embeddings):
    idx = indices.reshape(NW, NCHUNK, CB * SEQ).astype(jnp.int32)
    mesh = plsc.VectorSubcoreMesh(core_axis_name="c", subcore_axis_name="s")
    run = functools.partial(
        pl.kernel,
        mesh=mesh,
        out_type=jax.ShapeDtypeStruct((NW, NCHUNK, CB, SEQ, D), jnp.float32),
        scratch_types=[
            pltpu.VMEM((NCHUNK, CB * SEQ), jnp.int32),
            pltpu.VMEM((NB, CB * SEQ, D), jnp.float32),
        ] + [pltpu.SemaphoreType.DMA] * (2 * NB),
    )(_gather_kernel)
    out = run(embeddings, idx)
    return out.reshape(NBATCH, SEQ, D)


# speedup vs baseline: 2.3427x; 1.1329x over previous
import functools

import jax
import jax.numpy as jnp
from jax import lax
from jax.experimental import pallas as pl
from jax.experimental.pallas import tpu as pltpu
from jax.experimental.pallas import tpu_sc as plsc

VOCAB = 1_000_000
D = 128
SEQ = 50
NBATCH = 16384
NW = 32
BPW = NBATCH // NW
CB = 2
NCHUNK = BPW // CB
NB = 4
A = 3
NOUTER = NCHUNK // NB
assert NOUTER * NB == NCHUNK


def _gather_kernel(table_hbm, idx_hbm, out_hbm, idx_full, rows_v, *sems):
    gsems = sems[:NB]
    osems = sems[NB:]
    wid = lax.axis_index("s") * 2 + lax.axis_index("c")
    row0 = wid * BPW

    pltpu.sync_copy(idx_hbm.at[wid], idx_full)

    def start_gather(g, b):
        pltpu.make_async_copy(table_hbm.at[idx_full.at[g]], rows_v.at[b],
                              gsems[b]).start()

    def wait_gather(g, b):
        pltpu.make_async_copy(table_hbm.at[idx_full.at[g]], rows_v.at[b],
                              gsems[b]).wait()

    def start_out(g, b):
        for c in range(CB):
            pltpu.make_async_copy(rows_v.at[b, pl.ds(c * SEQ, SEQ)],
                                  out_hbm.at[row0 + g * CB + c],
                                  osems[b]).start()

    def wait_out(g, b):
        for c in range(CB):
            pltpu.make_async_copy(rows_v.at[b, pl.ds(c * SEQ, SEQ)],
                                  out_hbm.at[row0 + g * CB + c],
                                  osems[b]).wait()

    for g in range(A):
        start_gather(g, g % NB)

    def body(t, carry):
        for b0 in range(NB):
            g = t * NB + b0
            wait_gather(g, b0)
            start_out(g, b0)
            ga = g + A
            ba = (b0 + A) % NB

            @pl.when(ga < NCHUNK)
            def _issue_ahead():
                @pl.when(ga >= NB)
                def _reuse_guard():
                    wait_out(ga - NB, ba)
                start_gather(ga, ba)
        return carry

    lax.fori_loop(0, NOUTER, body, 0)

    for b in range(NB):
        g = NCHUNK - NB + b
        wait_out(g, g % NB)


def kernel(indices, embeddings):
    idx = indices.reshape(NW, NCHUNK, CB * SEQ).astype(jnp.int32)
    mesh = plsc.VectorSubcoreMesh(core_axis_name="c", subcore_axis_name="s")
    run = functools.partial(
        pl.kernel,
        mesh=mesh,
        out_type=jax.ShapeDtypeStruct((NBATCH, SEQ, D), jnp.float32),
        compiler_params=pltpu.CompilerParams(use_tc_tiling_on_sc=True),
        scratch_types=[
            pltpu.VMEM((NCHUNK, CB * SEQ), jnp.int32),
            pltpu.VMEM((NB, CB * SEQ, D), jnp.float32),
        ] + [pltpu.SemaphoreType.DMA] * (2 * NB),
    )(_gather_kernel)
    return run(embeddings, idx)
